# baseline (speedup 1.0000x reference)
import functools
import jax
import jax.numpy as jnp
from jax import lax
from jax.experimental import pallas as pl
from jax.experimental.pallas import tpu as pltpu
from jax.experimental.pallas import tpu_sc as plsc

NN = 10000
EE = 320000
D = 128
DE = 16
EH = 16
NEG = 0.2
F32 = jnp.float32

NC, NS, L = 2, 16, 16
NW = NC * NS
CHUNK = 64
TOTAL_CHUNKS = EE // CHUNK
CHUNKS_PW = TOTAL_CHUNKS // NW
CHUNKS_EXTRA = TOTAL_CHUNKS - CHUNKS_PW * NW
DW = 16
IB = 4
NBLK = CHUNKS_PW // IB
EUNR = 4
KD = D // L

ROWS_PT = 624
ROWS_TAIL = NN - ROWS_PT * NS


def _xlr(x, wl, wr):
    n = x.shape[0]
    b = 1000
    grid = n // b

    def body(x_ref, wl_ref, wr_ref, xl_ref, xr_ref):
        xb = x_ref[...]
        xl_ref[...] = jnp.dot(xb, wl_ref[...], preferred_element_type=F32)
        xr_ref[...] = jnp.dot(xb, wr_ref[...], preferred_element_type=F32)

    return pl.pallas_call(
        body,
        grid=(grid,),
        in_specs=[
            pl.BlockSpec((b, D), lambda i: (i, 0)),
            pl.BlockSpec((D, D), lambda i: (0, 0)),
            pl.BlockSpec((D, D), lambda i: (0, 0)),
        ],
        out_specs=[
            pl.BlockSpec((b, D), lambda i: (i, 0)),
            pl.BlockSpec((b, D), lambda i: (i, 0)),
        ],
        out_shape=[
            jax.ShapeDtypeStruct((n, D), F32),
            jax.ShapeDtypeStruct((n, D), F32),
        ],
    )(x, wl, wr)


def _ew(edge_attr, we, be_row, wae):
    b = 2000
    grid = EE // b

    def body(ea_ref, we_ref, be_ref, wae_ref, ew_ref):
        eh = jnp.dot(ea_ref[...], we_ref[...], preferred_element_type=F32)
        eh = eh + be_ref[...]
        ew_ref[...] = jnp.dot(eh, wae_ref[...], preferred_element_type=F32)

    return pl.pallas_call(
        body,
        grid=(grid,),
        in_specs=[
            pl.BlockSpec((b, DE), lambda i: (i, 0)),
            pl.BlockSpec((DE, EH), lambda i: (0, 0)),
            pl.BlockSpec((1, EH), lambda i: (0, 0)),
            pl.BlockSpec((EH, D), lambda i: (0, 0)),
        ],
        out_specs=pl.BlockSpec((b, D), lambda i: (i, 0)),
        out_shape=jax.ShapeDtypeStruct((EE, D), F32),
    )(edge_attr, we, be_row, wae)


def _combine_mm(acc, den_col, bias_row, wl, wr):
    b = 1000
    nb = NN // b

    def body(aa, ab, da, db, bias, wl_ref, wr_ref, xl_ref, xr_ref):
        d = da[...] + db[...] + 1e-16
        h = (aa[...] + ab[...]) / d + bias[...]
        h = jnp.maximum(h, 0.0)
        xl_ref[...] = jnp.dot(h, wl_ref[...], preferred_element_type=F32)
        xr_ref[...] = jnp.dot(h, wr_ref[...], preferred_element_type=F32)

    return pl.pallas_call(
        body,
        grid=(nb,),
        in_specs=[
            pl.BlockSpec((b, D), lambda i: (i, 0)),
            pl.BlockSpec((b, D), lambda i: (i + nb, 0)),
            pl.BlockSpec((b, 1), lambda i: (i, 0)),
            pl.BlockSpec((b, 1), lambda i: (i + nb, 0)),
            pl.BlockSpec((1, D), lambda i: (0, 0)),
            pl.BlockSpec((D, D), lambda i: (0, 0)),
            pl.BlockSpec((D, D), lambda i: (0, 0)),
        ],
        out_specs=[
            pl.BlockSpec((b, D), lambda i: (i, 0)),
            pl.BlockSpec((b, D), lambda i: (i, 0)),
        ],
        out_shape=[
            jax.ShapeDtypeStruct((NN, D), F32),
            jax.ShapeDtypeStruct((NN, D), F32),
        ],
    )(acc, acc, den_col, den_col, bias_row, wl, wr)


def _final(acc, den_col, bias_row):
    b = 1000
    nb = NN // b

    def body(aa, ab, da, db, bias, out_ref):
        d = da[...] + db[...] + 1e-16
        out_ref[...] = (aa[...] + ab[...]) / d + bias[...]

    return pl.pallas_call(
        body,
        grid=(nb,),
        in_specs=[
            pl.BlockSpec((b, D), lambda i: (i, 0)),
            pl.BlockSpec((b, D), lambda i: (i + nb, 0)),
            pl.BlockSpec((b, 1), lambda i: (i, 0)),
            pl.BlockSpec((b, 1), lambda i: (i + nb, 0)),
            pl.BlockSpec((1, D), lambda i: (0, 0)),
        ],
        out_specs=pl.BlockSpec((b, D), lambda i: (i, 0)),
        out_shape=jax.ShapeDtypeStruct((NN, D), F32),
    )(acc, acc, den_col, den_col, bias_row)


def _sc_body(xl_hbm, xr_hbm, ew_hbm, att_hbm, src_hbm, dst_hbm,
             zrow_hbm, zden_hbm, acc_out, den_out,
             xl_v0, xl_v1, xr_v, ew_v, w_v0, w_v1, src_ib, dst_ib,
             att_v, acc_sh, den_sh, gxl0, gxl1, gxe, ssem):
    cid = lax.axis_index("c")
    sid = lax.axis_index("s")
    wid = sid * NC + cid
    xl_bufs = (xl_v0, xl_v1)
    w_bufs = (w_v0, w_v1)
    gxl = (gxl0, gxl1)

    @pl.when(sid == 0)
    def _():
        pltpu.sync_copy(zrow_hbm, acc_sh)
        pltpu.sync_copy(zden_hbm, den_sh)

    pltpu.sync_copy(att_hbm, att_v)
    plsc.subcore_barrier()

    base_chunk = wid * CHUNKS_PW
    att_r = [att_v[k] for k in range(KD)]

    def compute(xl_b, w_b):
        def ebody(i, carry):
            for j in range(EUNR):
                e = i * EUNR + j
                avs = [xl_b[e, pl.ds(k * L, L)] for k in range(KD)]
                ps = []
                for k in range(KD):
                    s = avs[k] + xr_v[e, pl.ds(k * L, L)] \
                        + ew_v[e, pl.ds(k * L, L)]
                    ls = jnp.maximum(s, NEG * s)
                    ps.append(att_r[k] * ls)
                t0 = (ps[0] + ps[1]) + (ps[2] + ps[3])
                t1 = (ps[4] + ps[5]) + (ps[6] + ps[7])
                logit = jnp.sum(t0 + t1)
                wsp = jnp.exp(jnp.full((L,), logit, F32))
                w_b[e] = wsp
                for k in range(KD):
                    xl_b[e, pl.ds(k * L, L)] = wsp * avs[k]
            return carry

        lax.fori_loop(0, CHUNK // EUNR, ebody, 0)

    def idx_row(c):
        blk = c // IB
        return (blk % 2) * IB + (c - blk * IB)

    def fire_gxe(c):
        r = idx_row(c)
        pltpu.async_copy(xr_hbm.at[dst_ib.at[r]], xr_v, gxe)
        start = pl.multiple_of((base_chunk + c) * CHUNK, CHUNK)
        pltpu.async_copy(ew_hbm.at[pl.ds(start, CHUNK)], ew_v, gxe)

    def fire_xl(c, b):
        r = idx_row(c)
        pltpu.async_copy(xl_hbm.at[src_ib.at[r]], xl_bufs[b], gxl[b])

    def fire_scatter(c, b):
        r = idx_row(c)
        pltpu.async_copy(xl_bufs[b], acc_sh.at[dst_ib.at[r]], ssem, add=True)
        pltpu.async_copy(w_bufs[b], den_sh.at[dst_ib.at[r]], ssem, add=True)

    def drain_scatter():
        pltpu.make_async_copy(xl_hbm.at[pl.ds(0, CHUNK)], xl_v0, ssem).wait()
        pltpu.make_async_copy(den_out.at[pl.ds(0, CHUNK)], w_v0, ssem).wait()

    pltpu.sync_copy(src_hbm.at[pl.ds(base_chunk, IB)], src_ib.at[pl.ds(0, IB)])
    pltpu.sync_copy(dst_hbm.at[pl.ds(base_chunk, IB)], dst_ib.at[pl.ds(0, IB)])
    fire_xl(0, 0)
    fire_gxe(0)

    def step_body(s, carry):
        for b in range(2):
            c = s * 2 + b
            blk = c // IB
            pos = c - blk * IB

            @pl.when(jnp.logical_and(pos == 2, blk < NBLK - 1))
            def _():
                half = ((blk + 1) % 2) * IB
                row = base_chunk + (blk + 1) * IB
                pltpu.sync_copy(src_hbm.at[pl.ds(row, IB)],
                                src_ib.at[pl.ds(half, IB)])
                pltpu.sync_copy(dst_hbm.at[pl.ds(row, IB)],
                                dst_ib.at[pl.ds(half, IB)])

            pltpu.make_async_copy(xl_hbm.at[pl.ds(0, CHUNK)],
                                  xl_bufs[b], gxl[b]).wait()
            pltpu.make_async_copy(xr_hbm.at[pl.ds(0, CHUNK)],
                                  xr_v, gxe).wait()
            pltpu.make_async_copy(ew_hbm.at[pl.ds(0, CHUNK)],
                                  ew_v, gxe).wait()

            @pl.when(c >= 1)
            def _():
                drain_scatter()

            @pl.when(c < CHUNKS_PW - 1)
            def _():
                fire_xl(c + 1, 1 - b)

            compute(xl_bufs[b], w_bufs[b])

            @pl.when(c < CHUNKS_PW - 1)
            def _():
                fire_gxe(c + 1)

            fire_scatter(c, b)
        return carry

    lax.fori_loop(0, CHUNKS_PW // 2, step_body, 0)
    drain_scatter()

    @pl.when(wid < CHUNKS_EXTRA)
    def _():
        ec = NW * CHUNKS_PW + wid
        pltpu.sync_copy(src_hbm.at[pl.ds(ec, 1)], src_ib.at[pl.ds(0, 1)])
        pltpu.sync_copy(dst_hbm.at[pl.ds(ec, 1)], dst_ib.at[pl.ds(0, 1)])
        pltpu.async_copy(xl_hbm.at[src_ib.at[0]], xl_v0, gxl0).wait()
        pltpu.async_copy(xr_hbm.at[dst_ib.at[0]], xr_v, gxe).wait()
        start = pl.multiple_of(ec * CHUNK, CHUNK)
        pltpu.async_copy(ew_hbm.at[pl.ds(start, CHUNK)], ew_v, gxe).wait()
        compute(xl_v0, w_v0)
        pltpu.sync_copy(xl_v0, acc_sh.at[dst_ib.at[0]], add=True)
        pltpu.sync_copy(w_v0, den_sh.at[dst_ib.at[0]], add=True)

    plsc.subcore_barrier()

    r0 = sid * ROWS_PT
    o0 = cid * NN + r0
    pltpu.sync_copy(acc_sh.at[pl.ds(r0, ROWS_PT)], acc_out.at[pl.ds(o0, ROWS_PT)])
    pltpu.sync_copy(den_sh.at[pl.ds(r0, ROWS_PT)], den_out.at[pl.ds(o0, ROWS_PT)])

    @pl.when(sid == NS - 1)
    def _():
        rt = NS * ROWS_PT
        pltpu.sync_copy(acc_sh.at[pl.ds(rt, ROWS_TAIL)],
                        acc_out.at[pl.ds(cid * NN + rt, ROWS_TAIL)])
        pltpu.sync_copy(den_sh.at[pl.ds(rt, ROWS_TAIL)],
                        den_out.at[pl.ds(cid * NN + rt, ROWS_TAIL)])


_sc_layer = functools.partial(
    pl.kernel,
    out_type=[
        jax.ShapeDtypeStruct((NC * NN, D), F32),
        jax.ShapeDtypeStruct((NC * NN, DW), F32),
    ],
    mesh=plsc.VectorSubcoreMesh(core_axis_name="c", subcore_axis_name="s"),
    compiler_params=pltpu.CompilerParams(needs_layout_passes=False,
                                         use_tc_tiling_on_sc=False),
    scratch_types=[
        pltpu.VMEM((CHUNK, D), F32),
        pltpu.VMEM((CHUNK, D), F32),
        pltpu.VMEM((CHUNK, D), F32),
        pltpu.VMEM((CHUNK, D), F32),
        pltpu.VMEM((CHUNK, DW), F32),
        pltpu.VMEM((CHUNK, DW), F32),
        pltpu.VMEM((2 * IB, CHUNK), jnp.int32),
        pltpu.VMEM((2 * IB, CHUNK), jnp.int32),
        pltpu.VMEM((KD, L), F32),
        pltpu.VMEM_SHARED((NN, D), F32),
        pltpu.VMEM_SHARED((NN, DW), F32),
        pltpu.SemaphoreType.DMA,
        pltpu.SemaphoreType.DMA,
        pltpu.SemaphoreType.DMA,
        pltpu.SemaphoreType.DMA,
    ],
)(_sc_body)


def kernel(x, edge_index, edge_attr, We_emb, be_emb,
           Wl0, Wr0, Wae0, att0, b0,
           Wl1, Wr1, Wae1, att1, b1):
    src = edge_index[0].reshape(TOTAL_CHUNKS, CHUNK)
    dst = edge_index[1].reshape(TOTAL_CHUNKS, CHUNK)
    zrow = jnp.zeros((NN, D), F32)
    zden = jnp.zeros((NN, DW), F32)
    be_row = be_emb.reshape(1, EH)

    xl0, xr0 = _xlr(x, Wl0, Wr0)
    ew0 = _ew(edge_attr, We_emb, be_row, Wae0)
    ew1 = _ew(edge_attr, We_emb, be_row, Wae1)

    att0_b = att0.reshape(KD, L)
    att1_b = att1.reshape(KD, L)

    acc0, den0 = _sc_layer(xl0, xr0, ew0, att0_b, src, dst, zrow, zden)
    xl1, xr1 = _combine_mm(acc0, den0[:, :1], b0.reshape(1, D), Wl1, Wr1)

    acc1, den1 = _sc_layer(xl1, xr1, ew1, att1_b, src, dst, zrow, zden)
    return _final(acc1, den1[:, :1], b1.reshape(1, D))


# speedup vs baseline: 1.2052x; 1.2052x over previous
import functools
import jax
import jax.numpy as jnp
from jax import lax
from jax.experimental import pallas as pl
from jax.experimental.pallas import tpu as pltpu
from jax.experimental.pallas import tpu_sc as plsc

NN = 10000
EE = 320000
D = 128
DE = 16
EH = 16
NEG = 0.2
F32 = jnp.float32

NC, NS, L = 2, 16, 16
NW = NC * NS
CHUNK = 64
TOTAL_CHUNKS = EE // CHUNK
CHUNKS_PW = TOTAL_CHUNKS // NW
CHUNKS_EXTRA = TOTAL_CHUNKS - CHUNKS_PW * NW
DW = 16
IB = 12
NBLK = CHUNKS_PW // IB
EUNR = 2
HALF = CHUNK // 2
KD = D // L

ROWS_PT = 624
ROWS_TAIL = NN - ROWS_PT * NS


def _xlr(x, wl, wr):
    n = x.shape[0]
    b = 1000
    grid = n // b

    def body(x_ref, wl_ref, wr_ref, xl_ref, xr_ref):
        xb = x_ref[...]
        xl_ref[...] = jnp.dot(xb, wl_ref[...], preferred_element_type=F32)
        xr_ref[...] = jnp.dot(xb, wr_ref[...], preferred_element_type=F32)

    return pl.pallas_call(
        body,
        grid=(grid,),
        in_specs=[
            pl.BlockSpec((b, D), lambda i: (i, 0)),
            pl.BlockSpec((D, D), lambda i: (0, 0)),
            pl.BlockSpec((D, D), lambda i: (0, 0)),
        ],
        out_specs=[
            pl.BlockSpec((b, D), lambda i: (i, 0)),
            pl.BlockSpec((b, D), lambda i: (i, 0)),
        ],
        out_shape=[
            jax.ShapeDtypeStruct((n, D), F32),
            jax.ShapeDtypeStruct((n, D), F32),
        ],
    )(x, wl, wr)


def _ew(edge_attr, we, be_row, wae):
    b = 2000
    grid = EE // b

    def body(ea_ref, we_ref, be_ref, wae_ref, ew_ref):
        eh = jnp.dot(ea_ref[...], we_ref[...], preferred_element_type=F32)
        eh = eh + be_ref[...]
        ew_ref[...] = jnp.dot(eh, wae_ref[...], preferred_element_type=F32)

    return pl.pallas_call(
        body,
        grid=(grid,),
        in_specs=[
            pl.BlockSpec((b, DE), lambda i: (i, 0)),
            pl.BlockSpec((DE, EH), lambda i: (0, 0)),
            pl.BlockSpec((1, EH), lambda i: (0, 0)),
            pl.BlockSpec((EH, D), lambda i: (0, 0)),
        ],
        out_specs=pl.BlockSpec((b, D), lambda i: (i, 0)),
        out_shape=jax.ShapeDtypeStruct((EE, D), F32),
    )(edge_attr, we, be_row, wae)


def _combine_mm(acc, den_col, bias_row, wl, wr):
    b = 1000
    nb = NN // b

    def body(aa, ab, da, db, bias, wl_ref, wr_ref, xl_ref, xr_ref):
        d = da[...] + db[...] + 1e-16
        h = (aa[...] + ab[...]) / d + bias[...]
        h = jnp.maximum(h, 0.0)
        xl_ref[...] = jnp.dot(h, wl_ref[...], preferred_element_type=F32)
        xr_ref[...] = jnp.dot(h, wr_ref[...], preferred_element_type=F32)

    return pl.pallas_call(
        body,
        grid=(nb,),
        in_specs=[
            pl.BlockSpec((b, D), lambda i: (i, 0)),
            pl.BlockSpec((b, D), lambda i: (i + nb, 0)),
            pl.BlockSpec((b, 1), lambda i: (i, 0)),
            pl.BlockSpec((b, 1), lambda i: (i + nb, 0)),
            pl.BlockSpec((1, D), lambda i: (0, 0)),
            pl.BlockSpec((D, D), lambda i: (0, 0)),
            pl.BlockSpec((D, D), lambda i: (0, 0)),
        ],
        out_specs=[
            pl.BlockSpec((b, D), lambda i: (i, 0)),
            pl.BlockSpec((b, D), lambda i: (i, 0)),
        ],
        out_shape=[
            jax.ShapeDtypeStruct((NN, D), F32),
            jax.ShapeDtypeStruct((NN, D), F32),
        ],
    )(acc, acc, den_col, den_col, bias_row, wl, wr)


def _final(acc, den_col, bias_row):
    b = 1000
    nb = NN // b

    def body(aa, ab, da, db, bias, out_ref):
        d = da[...] + db[...] + 1e-16
        out_ref[...] = (aa[...] + ab[...]) / d + bias[...]

    return pl.pallas_call(
        body,
        grid=(nb,),
        in_specs=[
            pl.BlockSpec((b, D), lambda i: (i, 0)),
            pl.BlockSpec((b, D), lambda i: (i + nb, 0)),
            pl.BlockSpec((b, 1), lambda i: (i, 0)),
            pl.BlockSpec((b, 1), lambda i: (i + nb, 0)),
            pl.BlockSpec((1, D), lambda i: (0, 0)),
        ],
        out_specs=pl.BlockSpec((b, D), lambda i: (i, 0)),
        out_shape=jax.ShapeDtypeStruct((NN, D), F32),
    )(acc, acc, den_col, den_col, bias_row)


def _sc_body(xl_hbm, xr_hbm, ew_hbm, att_hbm, src_hbm, dst_hbm,
             zrow_hbm, zden_hbm, acc_out, den_out,
             xl_v0, xl_v1, xr_v, ew_v, w_v0, w_v1, src_ib, dst_ib,
             att_v, acc_sh, den_sh, gxl0, gxl1, gxe0, gxe1, ssem):
    cid = lax.axis_index("c")
    sid = lax.axis_index("s")
    wid = sid * NC + cid
    xl_bufs = (xl_v0, xl_v1)
    w_bufs = (w_v0, w_v1)
    gxl = (gxl0, gxl1)

    @pl.when(sid == 0)
    def _():
        pltpu.sync_copy(zrow_hbm, acc_sh)
        pltpu.sync_copy(zden_hbm, den_sh)

    pltpu.sync_copy(att_hbm, att_v)
    plsc.subcore_barrier()

    base_chunk = wid * CHUNKS_PW
    att_r = [att_v[k] for k in range(KD)]

    def compute(xl_b, w_b, h):
        def ebody(i, carry):
            for j in range(EUNR):
                e = h * HALF + i * EUNR + j
                avs = [xl_b[e, pl.ds(k * L, L)] for k in range(KD)]
                ps = []
                for k in range(KD):
                    s = avs[k] + xr_v[e, pl.ds(k * L, L)] \
                        + ew_v[e, pl.ds(k * L, L)]
                    ls = jnp.maximum(s, NEG * s)
                    ps.append(att_r[k] * ls)
                t0 = (ps[0] + ps[1]) + (ps[2] + ps[3])
                t1 = (ps[4] + ps[5]) + (ps[6] + ps[7])
                logit = jnp.sum(t0 + t1)
                wsp = jnp.exp(jnp.full((L,), logit, F32))
                w_b[e] = wsp
                for k in range(KD):
                    xl_b[e, pl.ds(k * L, L)] = wsp * avs[k]
            return carry

        lax.fori_loop(0, HALF // EUNR, ebody, 0)

    def idx_row(c):
        blk = c // IB
        return (blk % 2) * IB + (c - blk * IB)

    gxe = (gxe0, gxe1)

    def fire_gxe(c, h):
        r = idx_row(c)
        pltpu.async_copy(xr_hbm.at[dst_ib.at[r, pl.ds(h * HALF, HALF)]],
                         xr_v.at[pl.ds(h * HALF, HALF)], gxe[h])
        start = pl.multiple_of((base_chunk + c) * CHUNK + h * HALF, HALF)
        pltpu.async_copy(ew_hbm.at[pl.ds(start, HALF)],
                         ew_v.at[pl.ds(h * HALF, HALF)], gxe[h])

    def wait_gxe(h):
        pltpu.make_async_copy(xr_hbm.at[pl.ds(0, HALF)],
                              xr_v.at[pl.ds(h * HALF, HALF)], gxe[h]).wait()
        pltpu.make_async_copy(ew_hbm.at[pl.ds(0, HALF)],
                              ew_v.at[pl.ds(h * HALF, HALF)], gxe[h]).wait()

    def fire_xl(c, b):
        r = idx_row(c)
        pltpu.async_copy(xl_hbm.at[src_ib.at[r]], xl_bufs[b], gxl[b])

    def fire_scatter(c, b):
        r = idx_row(c)
        pltpu.async_copy(xl_bufs[b], acc_sh.at[dst_ib.at[r]], ssem, add=True)
        pltpu.async_copy(w_bufs[b], den_sh.at[dst_ib.at[r]], ssem, add=True)

    def drain_scatter():
        pltpu.make_async_copy(xl_hbm.at[pl.ds(0, CHUNK)], xl_v0, ssem).wait()
        pltpu.make_async_copy(den_out.at[pl.ds(0, CHUNK)], w_v0, ssem).wait()

    pltpu.sync_copy(src_hbm.at[pl.ds(base_chunk, IB)], src_ib.at[pl.ds(0, IB)])
    pltpu.sync_copy(dst_hbm.at[pl.ds(base_chunk, IB)], dst_ib.at[pl.ds(0, IB)])
    fire_xl(0, 0)
    fire_gxe(0, 0)
    fire_gxe(0, 1)

    def step_body(s, carry):
        for b in range(2):
            c = s * 2 + b
            blk = c // IB
            pos = c - blk * IB

            @pl.when(jnp.logical_and(pos == 2, blk < NBLK - 1))
            def _():
                half = ((blk + 1) % 2) * IB
                row = base_chunk + (blk + 1) * IB
                pltpu.sync_copy(src_hbm.at[pl.ds(row, IB)],
                                src_ib.at[pl.ds(half, IB)])
                pltpu.sync_copy(dst_hbm.at[pl.ds(row, IB)],
                                dst_ib.at[pl.ds(half, IB)])

            pltpu.make_async_copy(xl_hbm.at[pl.ds(0, CHUNK)],
                                  xl_bufs[b], gxl[b]).wait()

            @pl.when(c >= 1)
            def _():
                drain_scatter()

            @pl.when(c < CHUNKS_PW - 1)
            def _():
                fire_xl(c + 1, 1 - b)

            wait_gxe(0)
            compute(xl_bufs[b], w_bufs[b], 0)

            @pl.when(c < CHUNKS_PW - 1)
            def _():
                fire_gxe(c + 1, 0)

            wait_gxe(1)
            compute(xl_bufs[b], w_bufs[b], 1)

            @pl.when(c < CHUNKS_PW - 1)
            def _():
                fire_gxe(c + 1, 1)

            fire_scatter(c, b)
        return carry

    lax.fori_loop(0, CHUNKS_PW // 2, step_body, 0)
    drain_scatter()

    @pl.when(wid < CHUNKS_EXTRA)
    def _():
        ec = NW * CHUNKS_PW + wid
        pltpu.sync_copy(src_hbm.at[pl.ds(ec, 1)], src_ib.at[pl.ds(0, 1)])
        pltpu.sync_copy(dst_hbm.at[pl.ds(ec, 1)], dst_ib.at[pl.ds(0, 1)])
        pltpu.async_copy(xl_hbm.at[src_ib.at[0]], xl_v0, gxl0).wait()
        pltpu.async_copy(xr_hbm.at[dst_ib.at[0]], xr_v, gxe0).wait()
        start = pl.multiple_of(ec * CHUNK, CHUNK)
        pltpu.async_copy(ew_hbm.at[pl.ds(start, CHUNK)], ew_v, gxe0).wait()
        compute(xl_v0, w_v0, 0)
        compute(xl_v0, w_v0, 1)
        pltpu.sync_copy(xl_v0, acc_sh.at[dst_ib.at[0]], add=True)
        pltpu.sync_copy(w_v0, den_sh.at[dst_ib.at[0]], add=True)

    plsc.subcore_barrier()

    r0 = sid * ROWS_PT
    o0 = cid * NN + r0
    pltpu.sync_copy(acc_sh.at[pl.ds(r0, ROWS_PT)], acc_out.at[pl.ds(o0, ROWS_PT)])
    pltpu.sync_copy(den_sh.at[pl.ds(r0, ROWS_PT)], den_out.at[pl.ds(o0, ROWS_PT)])

    @pl.when(sid == NS - 1)
    def _():
        rt = NS * ROWS_PT
        pltpu.sync_copy(acc_sh.at[pl.ds(rt, ROWS_TAIL)],
                        acc_out.at[pl.ds(cid * NN + rt, ROWS_TAIL)])
        pltpu.sync_copy(den_sh.at[pl.ds(rt, ROWS_TAIL)],
                        den_out.at[pl.ds(cid * NN + rt, ROWS_TAIL)])


_sc_layer = functools.partial(
    pl.kernel,
    out_type=[
        jax.ShapeDtypeStruct((NC * NN, D), F32),
        jax.ShapeDtypeStruct((NC * NN, DW), F32),
    ],
    mesh=plsc.VectorSubcoreMesh(core_axis_name="c", subcore_axis_name="s"),
    compiler_params=pltpu.CompilerParams(needs_layout_passes=False,
                                         use_tc_tiling_on_sc=False),
    scratch_types=[
        pltpu.VMEM((CHUNK, D), F32),
        pltpu.VMEM((CHUNK, D), F32),
        pltpu.VMEM((CHUNK, D), F32),
        pltpu.VMEM((CHUNK, D), F32),
        pltpu.VMEM((CHUNK, DW), F32),
        pltpu.VMEM((CHUNK, DW), F32),
        pltpu.VMEM((2 * IB, CHUNK), jnp.int32),
        pltpu.VMEM((2 * IB, CHUNK), jnp.int32),
        pltpu.VMEM((KD, L), F32),
        pltpu.VMEM_SHARED((NN, D), F32),
        pltpu.VMEM_SHARED((NN, DW), F32),
        pltpu.SemaphoreType.DMA,
        pltpu.SemaphoreType.DMA,
        pltpu.SemaphoreType.DMA,
        pltpu.SemaphoreType.DMA,
        pltpu.SemaphoreType.DMA,
    ],
)(_sc_body)


def kernel(x, edge_index, edge_attr, We_emb, be_emb,
           Wl0, Wr0, Wae0, att0, b0,
           Wl1, Wr1, Wae1, att1, b1):
    src = edge_index[0].reshape(TOTAL_CHUNKS, CHUNK)
    dst = edge_index[1].reshape(TOTAL_CHUNKS, CHUNK)
    zrow = jnp.zeros((NN, D), F32)
    zden = jnp.zeros((NN, DW), F32)
    be_row = be_emb.reshape(1, EH)

    xl0, xr0 = _xlr(x, Wl0, Wr0)
    ew0 = _ew(edge_attr, We_emb, be_row, Wae0)
    ew1 = _ew(edge_attr, We_emb, be_row, Wae1)

    att0_b = att0.reshape(KD, L)
    att1_b = att1.reshape(KD, L)

    acc0, den0 = _sc_layer(xl0, xr0, ew0, att0_b, src, dst, zrow, zden)
    xl1, xr1 = _combine_mm(acc0, den0[:, :1], b0.reshape(1, D), Wl1, Wr1)

    acc1, den1 = _sc_layer(xl1, xr1, ew1, att1_b, src, dst, zrow, zden)
    return _final(acc1, den1[:, :1], b1.reshape(1, D))
